# baseline (speedup 1.0000x reference)
import jax
import jax.numpy as jnp
from jax import lax
from jax.experimental import pallas as pl
from jax.experimental.pallas import tpu as pltpu

B = 128
S = 32768
RB = 4
NCHUNK = B // RB
NBUF = 6


def _body(idx_ref, logits_hbm, mask_hbm, out_l_hbm, out_m_hbm,
          lbuf, mbuf, olbuf, insem, outsem):
    def in_copy(i, slot):
        return pltpu.make_async_copy(
            logits_hbm.at[pl.ds(i * RB, RB), :], lbuf.at[slot], insem.at[slot, 0]
        ), pltpu.make_async_copy(
            mask_hbm.at[pl.ds(i * RB, RB), :], mbuf.at[slot], insem.at[slot, 1]
        )

    def out_copy(i, slot):
        return pltpu.make_async_copy(
            olbuf.at[slot], out_l_hbm.at[pl.ds(i * RB, RB), :], outsem.at[slot, 0]
        ), pltpu.make_async_copy(
            mbuf.at[slot], out_m_hbm.at[pl.ds(i * RB, RB), :], outsem.at[slot, 1]
        )

    for i in range(NBUF):
        a, b = in_copy(i, i)
        a.start()
        b.start()

    lane = lax.broadcasted_iota(jnp.int32, (1, 128), 1)

    def step(i, _):
        slot = lax.rem(i, NBUF)
        a, b = in_copy(i, slot)
        a.wait()
        b.wait()
        @pl.when(i >= NBUF)
        def _():
            c, d = out_copy(i - NBUF, slot)
            c.wait()
            d.wait()

        for r in range(RB):
            idx = idx_ref[i * RB + r]
            c0 = pl.multiple_of(idx & ~127, 128)
            seg = mbuf[slot, pl.ds(r, 1), pl.ds(c0, 128)]
            mbuf[slot, pl.ds(r, 1), pl.ds(c0, 128)] = jnp.where(
                lane == idx - c0, jnp.uint8(1), seg
            )

        olbuf[slot] = jnp.where(mbuf[slot] != 0, -jnp.inf, lbuf[slot])
        c, d = out_copy(i, slot)
        c.start()
        d.start()

        @pl.when(i + NBUF < NCHUNK)
        def _():
            a2, b2 = in_copy(i + NBUF, slot)
            a2.start()
            b2.start()

        return 0

    lax.fori_loop(0, NCHUNK, step, 0, unroll=False)

    for i in range(NCHUNK - NBUF, NCHUNK):
        c, d = out_copy(i, i % NBUF)
        c.wait()
        d.wait()


def kernel(logits, mask, idxs):
    out_l, out_m = pl.pallas_call(
        _body,
        in_specs=[
            pl.BlockSpec(memory_space=pltpu.SMEM),
            pl.BlockSpec(memory_space=pl.ANY),
            pl.BlockSpec(memory_space=pl.ANY),
        ],
        out_specs=[
            pl.BlockSpec(memory_space=pl.ANY),
            pl.BlockSpec(memory_space=pl.ANY),
        ],
        out_shape=[
            jax.ShapeDtypeStruct((B, S), jnp.float32),
            jax.ShapeDtypeStruct((B, S), jnp.uint8),
        ],
        compiler_params=pltpu.CompilerParams(skip_device_barrier=True),
        scratch_shapes=[
            pltpu.VMEM((NBUF, RB, S), jnp.float32),
            pltpu.VMEM((NBUF, RB, S), jnp.uint8),
            pltpu.VMEM((NBUF, RB, S), jnp.float32),
            pltpu.SemaphoreType.DMA((NBUF, 2)),
            pltpu.SemaphoreType.DMA((NBUF, 2)),
        ],
    )(idxs.astype(jnp.int32), logits, mask.view(jnp.uint8))
    return out_l, out_m.view(jnp.bool_)


# speedup vs baseline: 1.0234x; 1.0234x over previous
import jax
import jax.numpy as jnp
from jax import lax
from jax.experimental import pallas as pl
from jax.experimental.pallas import tpu as pltpu

B = 128
S = 32768
RB = 4
NCHUNK = B // RB
NBUF = 8


def _body(idx_ref, logits_hbm, mask_hbm, out_l_hbm, out_m_hbm,
          lbuf, mbuf, olbuf, insem, outsem):
    def in_copy(i, slot):
        return pltpu.make_async_copy(
            logits_hbm.at[pl.ds(i * RB, RB), :], lbuf.at[slot], insem.at[slot, 0]
        ), pltpu.make_async_copy(
            mask_hbm.at[pl.ds(i * RB, RB), :], mbuf.at[slot], insem.at[slot, 1]
        )

    def out_copy(i, slot):
        return pltpu.make_async_copy(
            olbuf.at[slot], out_l_hbm.at[pl.ds(i * RB, RB), :], outsem.at[slot, 0]
        ), pltpu.make_async_copy(
            mbuf.at[slot], out_m_hbm.at[pl.ds(i * RB, RB), :], outsem.at[slot, 1]
        )

    for i in range(NBUF):
        a, b = in_copy(i, i)
        a.start()
        b.start()

    lane = lax.broadcasted_iota(jnp.int32, (1, 128), 1)

    def step(i, _):
        slot = lax.rem(i, NBUF)
        a, b = in_copy(i, slot)
        a.wait()
        b.wait()
        @pl.when(i >= NBUF)
        def _():
            c, d = out_copy(i - NBUF, slot)
            c.wait()
            d.wait()

        for r in range(RB):
            idx = idx_ref[i * RB + r]
            c0 = pl.multiple_of(idx & ~127, 128)
            seg = mbuf[slot, pl.ds(r, 1), pl.ds(c0, 128)]
            mbuf[slot, pl.ds(r, 1), pl.ds(c0, 128)] = jnp.where(
                lane == idx - c0, jnp.uint8(1), seg
            )

        olbuf[slot] = jnp.where(mbuf[slot] != 0, -jnp.inf, lbuf[slot])
        c, d = out_copy(i, slot)
        c.start()
        d.start()

        @pl.when(i + NBUF < NCHUNK)
        def _():
            a2, b2 = in_copy(i + NBUF, slot)
            a2.start()
            b2.start()

        return 0

    lax.fori_loop(0, NCHUNK, step, 0, unroll=False)

    for i in range(NCHUNK - NBUF, NCHUNK):
        c, d = out_copy(i, i % NBUF)
        c.wait()
        d.wait()


def kernel(logits, mask, idxs):
    out_l, out_m = pl.pallas_call(
        _body,
        in_specs=[
            pl.BlockSpec(memory_space=pltpu.SMEM),
            pl.BlockSpec(memory_space=pl.ANY),
            pl.BlockSpec(memory_space=pl.ANY),
        ],
        out_specs=[
            pl.BlockSpec(memory_space=pl.ANY),
            pl.BlockSpec(memory_space=pl.ANY),
        ],
        out_shape=[
            jax.ShapeDtypeStruct((B, S), jnp.float32),
            jax.ShapeDtypeStruct((B, S), jnp.uint8),
        ],
        scratch_shapes=[
            pltpu.VMEM((NBUF, RB, S), jnp.float32),
            pltpu.VMEM((NBUF, RB, S), jnp.uint8),
            pltpu.VMEM((NBUF, RB, S), jnp.float32),
            pltpu.SemaphoreType.DMA((NBUF, 2)),
            pltpu.SemaphoreType.DMA((NBUF, 2)),
        ],
    )(idxs.astype(jnp.int32), logits, mask.view(jnp.uint8))
    return out_l, out_m.view(jnp.bool_)


# speedup vs baseline: 1.0304x; 1.0068x over previous
import jax
import jax.numpy as jnp
from jax import lax
from jax.experimental import pallas as pl
from jax.experimental.pallas import tpu as pltpu

B = 128
S = 32768
RB = 4
NCHUNK = B // RB
NBUF = 12


def _body(idx_ref, logits_hbm, mask_hbm, out_l_hbm, out_m_hbm,
          lbuf, mbuf, olbuf, insem, outsem):
    def in_copy(i, slot):
        return pltpu.make_async_copy(
            logits_hbm.at[pl.ds(i * RB, RB), :], lbuf.at[slot], insem.at[slot, 0]
        ), pltpu.make_async_copy(
            mask_hbm.at[pl.ds(i * RB, RB), :], mbuf.at[slot], insem.at[slot, 1]
        )

    def out_copy(i, slot):
        return pltpu.make_async_copy(
            olbuf.at[slot], out_l_hbm.at[pl.ds(i * RB, RB), :], outsem.at[slot, 0]
        ), pltpu.make_async_copy(
            mbuf.at[slot], out_m_hbm.at[pl.ds(i * RB, RB), :], outsem.at[slot, 1]
        )

    for i in range(NBUF):
        a, b = in_copy(i, i)
        a.start()
        b.start()

    lane = lax.broadcasted_iota(jnp.int32, (1, 128), 1)

    def step(i, _):
        slot = lax.rem(i, NBUF)
        a, b = in_copy(i, slot)
        a.wait()
        b.wait()
        @pl.when(i >= NBUF)
        def _():
            c, d = out_copy(i - NBUF, slot)
            c.wait()
            d.wait()

        for r in range(RB):
            idx = idx_ref[i * RB + r]
            c0 = pl.multiple_of(idx & ~127, 128)
            seg = mbuf[slot, pl.ds(r, 1), pl.ds(c0, 128)]
            mbuf[slot, pl.ds(r, 1), pl.ds(c0, 128)] = jnp.where(
                lane == idx - c0, jnp.uint8(1), seg
            )

        olbuf[slot] = jnp.where(mbuf[slot] != 0, -jnp.inf, lbuf[slot])
        c, d = out_copy(i, slot)
        c.start()
        d.start()

        @pl.when(i + NBUF < NCHUNK)
        def _():
            a2, b2 = in_copy(i + NBUF, slot)
            a2.start()
            b2.start()

        return 0

    lax.fori_loop(0, NCHUNK, step, 0, unroll=False)

    for i in range(NCHUNK - NBUF, NCHUNK):
        c, d = out_copy(i, i % NBUF)
        c.wait()
        d.wait()


def kernel(logits, mask, idxs):
    out_l, out_m = pl.pallas_call(
        _body,
        in_specs=[
            pl.BlockSpec(memory_space=pltpu.SMEM),
            pl.BlockSpec(memory_space=pl.ANY),
            pl.BlockSpec(memory_space=pl.ANY),
        ],
        out_specs=[
            pl.BlockSpec(memory_space=pl.ANY),
            pl.BlockSpec(memory_space=pl.ANY),
        ],
        out_shape=[
            jax.ShapeDtypeStruct((B, S), jnp.float32),
            jax.ShapeDtypeStruct((B, S), jnp.uint8),
        ],
        scratch_shapes=[
            pltpu.VMEM((NBUF, RB, S), jnp.float32),
            pltpu.VMEM((NBUF, RB, S), jnp.uint8),
            pltpu.VMEM((NBUF, RB, S), jnp.float32),
            pltpu.SemaphoreType.DMA((NBUF, 2)),
            pltpu.SemaphoreType.DMA((NBUF, 2)),
        ],
    )(idxs.astype(jnp.int32), logits, mask.view(jnp.uint8))
    return out_l, out_m.view(jnp.bool_)
